# baseline (speedup 1.0000x reference)
import functools

import jax
import jax.numpy as jnp
from jax import lax
from jax.experimental import pallas as pl
from jax.experimental.pallas import tpu as pltpu
from jax.experimental.pallas import tpu_sc as plsc

_C = 5
_L = 16
_NC = 2
_NS = 16
_NW = _NC * _NS

_LN2 = 0.6931471805599453
_RK = 0x3F3504F3


def _vlog(x):
    bits = lax.bitcast_convert_type(x, jnp.int32)
    tmp = bits - _RK
    e = lax.shift_right_arithmetic(tmp, 23)
    m = lax.bitcast_convert_type((tmp & 0x007FFFFF) + _RK, jnp.float32)
    z = (m - 1.0) / (m + 1.0)
    z2 = z * z
    p = jnp.float32(2.0 / 11.0)
    for c in (2.0 / 9.0, 2.0 / 7.0, 2.0 / 5.0, 2.0 / 3.0, 2.0):
        p = p * z2 + jnp.float32(c)
    return e.astype(jnp.float32) * jnp.float32(_LN2) + z * p


def _butterfly(x, op):
    ii = lax.iota(jnp.int32, _L)
    for sh in (8, 4, 2, 1):
        x = op(x, x.at[ii ^ sh].get(mode="promise_in_bounds"))
    return x


def _gumbel_argmax(logits, u):
    score = logits + (-_vlog(-_vlog(u)))
    mx = _butterfly(score, jnp.maximum)
    ii = lax.iota(jnp.int32, _L)
    cand = jnp.where(score == mx, ii, jnp.int32(_L))
    return _butterfly(cand, jnp.minimum)


def _make_sc_kernel(total, chunk):
    per_w = total // _NW
    steps = per_w // chunk
    mesh = plsc.VectorSubcoreMesh(core_axis_name="c", subcore_axis_name="s")

    @functools.partial(
        pl.kernel,
        mesh=mesh,
        out_type=jax.ShapeDtypeStruct((total,), jnp.float32),
        scratch_types=[
            pltpu.VMEM((_L,), jnp.float32),
            pltpu.VMEM((_L,), jnp.float32),
            pltpu.VMEM((_L,), jnp.float32),
            pltpu.VMEM((_L,), jnp.float32),
            pltpu.VMEM((chunk,), jnp.float32),
        ],
    )
    def k(pm_hbm, ps_hbm, um_hbm, us_hbm, eps_hbm, out_hbm,
          pm_v, ps_v, um_v, us_v, buf):
        pltpu.sync_copy(pm_hbm, pm_v)
        pltpu.sync_copy(ps_hbm, ps_v)
        pltpu.sync_copy(um_hbm, um_v)
        pltpu.sync_copy(us_hbm, us_v)

        mu_idx = _gumbel_argmax(pm_v[...], um_v[...])
        sig_idx = _gumbel_argmax(ps_v[...], us_v[...])
        a_vec = mu_idx.astype(jnp.float32)
        b_vec = jnp.exp(sig_idx.astype(jnp.float32))

        wid = lax.axis_index("s") * _NC + lax.axis_index("c")
        base = pl.multiple_of(wid * per_w, chunk)

        def step(g, carry):
            off = pl.multiple_of(base + g * chunk, chunk)
            pltpu.sync_copy(eps_hbm.at[pl.ds(off, chunk)], buf)

            def inner(i, c):
                s = pl.multiple_of(i * _L, _L)
                buf[pl.ds(s, _L)] = a_vec + b_vec * buf[pl.ds(s, _L)]
                return c

            lax.fori_loop(0, chunk // _L, inner, 0, unroll=8)
            pltpu.sync_copy(buf, out_hbm.at[pl.ds(off, chunk)])
            return carry

        lax.fori_loop(0, steps, step, 0)

    return k


def kernel(prob_mu, prob_sigma, u_mu, u_sigma, eps):
    n = eps.shape[0]
    total = n * _C

    def pad16(v, fill):
        return jnp.concatenate(
            [v, jnp.full((_L - _C,), fill, v.dtype)])

    pm16 = pad16(prob_mu, -1e30)
    ps16 = pad16(prob_sigma, -1e30)
    um16 = pad16(u_mu, 0.5)
    us16 = pad16(u_sigma, 0.5)

    k = _make_sc_kernel(total, chunk=16384)
    flat = k(pm16, ps16, um16, us16, eps.reshape(total))
    return flat.reshape(n, _C)


# speedup vs baseline: 40.7955x; 40.7955x over previous
import functools

import jax
import jax.numpy as jnp
from jax import lax
from jax.experimental import pallas as pl
from jax.experimental.pallas import tpu as pltpu

_C = 5
_LANES = 128


def _gumbel_argmax(logits, u):
    score = logits - jnp.log(-jnp.log(u))
    mx = jnp.max(score)
    ii = lax.broadcasted_iota(jnp.int32, (1, _LANES), 1)
    return jnp.min(jnp.where(score == mx, ii, _LANES))


def _body(pm_ref, ps_ref, um_ref, us_ref, x_ref, o_ref):
    mu_idx = _gumbel_argmax(pm_ref[...], um_ref[...])
    sig_idx = _gumbel_argmax(ps_ref[...], us_ref[...])
    a = mu_idx.astype(jnp.float32)
    b = jnp.exp(sig_idx.astype(jnp.float32))
    o_ref[...] = a + b * x_ref[...]


@functools.partial(jax.jit, static_argnames=("w",))
def _run(pm, ps, um, us, x_t, w):
    n = x_t.shape[1]
    grid = (n // w,)
    param_spec = pl.BlockSpec((1, _LANES), lambda i: (0, 0))
    return pl.pallas_call(
        _body,
        grid=grid,
        in_specs=[
            param_spec, param_spec, param_spec, param_spec,
            pl.BlockSpec((_C, w), lambda i: (0, i)),
        ],
        out_specs=pl.BlockSpec((_C, w), lambda i: (0, i)),
        out_shape=jax.ShapeDtypeStruct((_C, n), jnp.float32),
        compiler_params=pltpu.CompilerParams(
            dimension_semantics=("arbitrary",),
        ),
    )(pm, ps, um, us, x_t)


def kernel(prob_mu, prob_sigma, u_mu, u_sigma, eps):
    n = eps.shape[0]

    def pad128(v, fill):
        return jnp.concatenate(
            [v, jnp.full((_LANES - _C,), fill, v.dtype)]).reshape(1, _LANES)

    pm = pad128(prob_mu, -1e30)
    ps = pad128(prob_sigma, -1e30)
    um = pad128(u_mu, 0.5)
    us = pad128(u_sigma, 0.5)

    out_t = _run(pm, ps, um, us, eps.T, w=65536)
    return out_t.T


# speedup vs baseline: 50.8684x; 1.2469x over previous
import functools

import jax
import jax.numpy as jnp
from jax import lax
from jax.experimental import pallas as pl
from jax.experimental.pallas import tpu as pltpu

_C = 5
_LANES = 128


def _gumbel_argmax(logits, u):
    score = logits - jnp.log(-jnp.log(u))
    mx = jnp.max(score)
    ii = lax.broadcasted_iota(jnp.int32, (1, _LANES), 1)
    return jnp.min(jnp.where(score == mx, ii, _LANES))


def _body(pm_ref, ps_ref, um_ref, us_ref, x_ref, o_ref):
    mu_idx = _gumbel_argmax(pm_ref[...], um_ref[...])
    sig_idx = _gumbel_argmax(ps_ref[...], us_ref[...])
    a = mu_idx.astype(jnp.float32)
    b = jnp.exp(sig_idx.astype(jnp.float32))
    o_ref[...] = a + b * x_ref[...]


@functools.partial(jax.jit, static_argnames=("w",))
def _run(pm, ps, um, us, x_t, w):
    n = x_t.shape[1]
    grid = (n // w,)
    param_spec = pl.BlockSpec((1, _LANES), lambda i: (0, 0))
    return pl.pallas_call(
        _body,
        grid=grid,
        in_specs=[
            param_spec, param_spec, param_spec, param_spec,
            pl.BlockSpec((_C, w), lambda i: (0, i)),
        ],
        out_specs=pl.BlockSpec((_C, w), lambda i: (0, i)),
        out_shape=jax.ShapeDtypeStruct((_C, n), jnp.float32),
        compiler_params=pltpu.CompilerParams(
            dimension_semantics=("arbitrary",),
        ),
    )(pm, ps, um, us, x_t)


def kernel(prob_mu, prob_sigma, u_mu, u_sigma, eps):
    n = eps.shape[0]

    def pad128(v, fill):
        return jnp.concatenate(
            [v, jnp.full((_LANES - _C,), fill, v.dtype)]).reshape(1, _LANES)

    pm = pad128(prob_mu, -1e30)
    ps = pad128(prob_sigma, -1e30)
    um = pad128(u_mu, 0.5)
    us = pad128(u_sigma, 0.5)

    out_t = _run(pm, ps, um, us, eps.T, w=131072)
    return out_t.T


# speedup vs baseline: 52.7905x; 1.0378x over previous
import functools

import jax
import jax.numpy as jnp
from jax import lax
from jax.experimental import pallas as pl
from jax.experimental.pallas import tpu as pltpu

_C = 5
_LANES = 128


def _gumbel_argmax(logits, u):
    score = logits - jnp.log(-jnp.log(u))
    mx = jnp.max(score)
    ii = lax.broadcasted_iota(jnp.int32, (1, _LANES), 1)
    return jnp.min(jnp.where(score == mx, ii, _LANES))


def _body(pm_ref, ps_ref, um_ref, us_ref, x_ref, o_ref):
    mu_idx = _gumbel_argmax(pm_ref[...], um_ref[...])
    sig_idx = _gumbel_argmax(ps_ref[...], us_ref[...])
    a = mu_idx.astype(jnp.float32)
    b = jnp.exp(sig_idx.astype(jnp.float32))
    o_ref[...] = a + b * x_ref[...]


@functools.partial(jax.jit, static_argnames=("w",))
def _run(pm, ps, um, us, x_t, w):
    n = x_t.shape[1]
    grid = (n // w,)
    param_spec = pl.BlockSpec((1, _LANES), lambda i: (0, 0))
    return pl.pallas_call(
        _body,
        grid=grid,
        in_specs=[
            param_spec, param_spec, param_spec, param_spec,
            pl.BlockSpec((_C, w), lambda i: (0, i)),
        ],
        out_specs=pl.BlockSpec((_C, w), lambda i: (0, i)),
        out_shape=jax.ShapeDtypeStruct((_C, n), jnp.float32),
        compiler_params=pltpu.CompilerParams(
            dimension_semantics=("arbitrary",),
        ),
    )(pm, ps, um, us, x_t)


def kernel(prob_mu, prob_sigma, u_mu, u_sigma, eps):
    n = eps.shape[0]

    def pad128(v, fill):
        return jnp.concatenate(
            [v, jnp.full((_LANES - _C,), fill, v.dtype)]).reshape(1, _LANES)

    pm = pad128(prob_mu, -1e30)
    ps = pad128(prob_sigma, -1e30)
    um = pad128(u_mu, 0.5)
    us = pad128(u_sigma, 0.5)

    out_t = _run(pm, ps, um, us, eps.T, w=262144)
    return out_t.T


# speedup vs baseline: 53.3015x; 1.0097x over previous
import functools

import jax
import jax.numpy as jnp
from jax import lax
from jax.experimental import pallas as pl
from jax.experimental.pallas import tpu as pltpu

_C = 5
_LANES = 128


def _gumbel_argmax(logits, u):
    score = logits - jnp.log(-jnp.log(u))
    mx = jnp.max(score)
    ii = lax.broadcasted_iota(jnp.int32, (1, _LANES), 1)
    return jnp.min(jnp.where(score == mx, ii, _LANES))


def _body(pm_ref, ps_ref, um_ref, us_ref, x_ref, o_ref):
    mu_idx = _gumbel_argmax(pm_ref[...], um_ref[...])
    sig_idx = _gumbel_argmax(ps_ref[...], us_ref[...])
    a = mu_idx.astype(jnp.float32)
    b = jnp.exp(sig_idx.astype(jnp.float32))
    o_ref[...] = a + b * x_ref[...]


def _manual_body(d, w2, steps):
    def body(pm_ref, ps_ref, um_ref, us_ref, x_hbm, o_hbm,
             inb, outb, insem, outsem):
        mu_idx = _gumbel_argmax(pm_ref[...], um_ref[...])
        sig_idx = _gumbel_argmax(ps_ref[...], us_ref[...])
        a = mu_idx.astype(jnp.float32)
        b = jnp.exp(sig_idx.astype(jnp.float32))

        def in_copy(off, s):
            return pltpu.make_async_copy(
                x_hbm.at[:, pl.ds(off, w2)], inb.at[s], insem.at[s])

        def out_copy(off, s):
            return pltpu.make_async_copy(
                outb.at[s], o_hbm.at[:, pl.ds(off, w2)], outsem.at[s])

        for s in range(d):
            in_copy(s * w2, s).start()

        def loop(g2, carry):
            for s in range(d):
                g = g2 * d + s
                off = pl.multiple_of(g * w2, w2)
                in_copy(off, s).wait()

                @pl.when(g2 > 0)
                def _wait_prev_out():
                    out_copy(0, s).wait()

                outb[s] = a + b * inb[s]
                out_copy(off, s).start()

                @pl.when(g + d < steps)
                def _start_next_in():
                    nxt = pl.multiple_of((g + d) * w2, w2)
                    in_copy(nxt, s).start()
            return carry

        lax.fori_loop(0, steps // d, loop, 0)
        for s in range(d):
            out_copy((steps - d + s) * w2, s).wait()

    return body


@functools.partial(jax.jit, static_argnames=("d", "w2"))
def _run_manual(pm, ps, um, us, x_t, d, w2):
    n = x_t.shape[1]
    steps = n // w2
    return pl.pallas_call(
        _manual_body(d, w2, steps),
        in_specs=[
            pl.BlockSpec(memory_space=pltpu.MemorySpace.VMEM),
            pl.BlockSpec(memory_space=pltpu.MemorySpace.VMEM),
            pl.BlockSpec(memory_space=pltpu.MemorySpace.VMEM),
            pl.BlockSpec(memory_space=pltpu.MemorySpace.VMEM),
            pl.BlockSpec(memory_space=pltpu.MemorySpace.HBM),
        ],
        out_specs=pl.BlockSpec(memory_space=pltpu.MemorySpace.HBM),
        out_shape=jax.ShapeDtypeStruct((_C, n), jnp.float32),
        scratch_shapes=[
            pltpu.VMEM((d, _C, w2), jnp.float32),
            pltpu.VMEM((d, _C, w2), jnp.float32),
            pltpu.SemaphoreType.DMA((d,)),
            pltpu.SemaphoreType.DMA((d,)),
        ],
    )(pm, ps, um, us, x_t)


@functools.partial(jax.jit, static_argnames=("w",))
def _run(pm, ps, um, us, x_t, w):
    n = x_t.shape[1]
    grid = (n // w,)
    param_spec = pl.BlockSpec((1, _LANES), lambda i: (0, 0))
    return pl.pallas_call(
        _body,
        grid=grid,
        in_specs=[
            param_spec, param_spec, param_spec, param_spec,
            pl.BlockSpec((_C, w), lambda i: (0, i)),
        ],
        out_specs=pl.BlockSpec((_C, w), lambda i: (0, i)),
        out_shape=jax.ShapeDtypeStruct((_C, n), jnp.float32),
        compiler_params=pltpu.CompilerParams(
            dimension_semantics=("arbitrary",),
        ),
    )(pm, ps, um, us, x_t)


def kernel(prob_mu, prob_sigma, u_mu, u_sigma, eps):
    n = eps.shape[0]

    def pad128(v, fill):
        return jnp.concatenate(
            [v, jnp.full((_LANES - _C,), fill, v.dtype)]).reshape(1, _LANES)

    pm = pad128(prob_mu, -1e30)
    ps = pad128(prob_sigma, -1e30)
    um = pad128(u_mu, 0.5)
    us = pad128(u_sigma, 0.5)

    out_t = _run_manual(pm, ps, um, us, eps.T, d=4, w2=131072)
    return out_t.T
